# baseline (speedup 1.0000x reference)
import jax
import jax.numpy as jnp
from jax import lax
from jax.experimental import pallas as pl
from jax.experimental.pallas import tpu as pltpu
from jax.experimental.pallas import tpu_sc as plsc

NUM_FEATURES = 7
NUM_BINS = 10
EMB_DIM = 18
BATCH = 16384
NUM_INNER = NUM_BINS - 1

NC = 2
NS = 16
NW = NC * NS
LANES = 16

ROWS_PER_W = BATCH // NW
FLAT_PER_W = ROWS_PER_W * NUM_FEATURES
OUT_PER_W = FLAT_PER_W * EMB_DIM
GROUPS = ROWS_PER_W // LANES
TABLE_SIZE = NUM_FEATURES * NUM_BINS * EMB_DIM
BOUNDS_SIZE = NUM_FEATURES * NUM_INNER * LANES
MS_SIZE = NUM_FEATURES * LANES


def _sc_body(raw_hbm, bounds_hbm, means_hbm, stds_hbm, table_hbm, out_hbm,
             raw_v, bounds_v, means_v, stds_v, table_v, rows_v):
    wid = lax.axis_index("s") * NC + lax.axis_index("c")
    base = wid * FLAT_PER_W

    pltpu.sync_copy(raw_hbm.at[pl.ds(base, FLAT_PER_W)], raw_v)
    pltpu.sync_copy(bounds_hbm, bounds_v)
    pltpu.sync_copy(means_hbm, means_v)
    pltpu.sync_copy(stds_hbm, stds_v)
    pltpu.sync_copy(table_hbm, table_v)

    lane = lax.broadcasted_iota(jnp.int32, (LANES,), 0)
    lane_f = lane * NUM_FEATURES
    lane_fe = lane_f * EMB_DIM

    for i in range(NUM_FEATURES):
        mean_i = means_v[pl.ds(i * LANES, LANES)]
        std_i = stds_v[pl.ds(i * LANES, LANES)]
        bvecs = [bounds_v[pl.ds((i * NUM_INNER + k) * LANES, LANES)]
                 for k in range(NUM_INNER)]

        @plsc.parallel_loop(0, GROUPS, 1, unroll=4)
        def _(g, i=i, mean_i=mean_i, std_i=std_i, bvecs=bvecs):
            flat0 = g * (LANES * NUM_FEATURES) + i
            gidx = lane_f + flat0
            vals = plsc.load_gather(raw_v, [gidx])
            x = (vals - mean_i) / (std_i + 1e-8)
            cnt = jnp.zeros((LANES,), jnp.int32)
            for k in range(NUM_INNER):
                cnt = cnt + (x > bvecs[k]).astype(jnp.int32)
            src0 = (cnt + i * NUM_BINS) * EMB_DIM
            dst0 = lane_fe + flat0 * EMB_DIM
            for d in range(EMB_DIM):
                e = plsc.load_gather(table_v, [src0 + d])
                plsc.store_scatter(rows_v, [dst0 + d], e)

    pltpu.sync_copy(rows_v, out_hbm.at[pl.ds(wid * OUT_PER_W, OUT_PER_W)])


@jax.jit
def _encode(raw_flat, bounds_b, means_b, stds_b, table_flat):
    mesh = plsc.VectorSubcoreMesh(
        core_axis_name="c", subcore_axis_name="s",
        num_cores=NC, num_subcores=NS,
    )
    return pl.kernel(
        _sc_body,
        out_type=jax.ShapeDtypeStruct((BATCH * NUM_FEATURES * EMB_DIM,),
                                      jnp.float32),
        mesh=mesh,
        compiler_params=pltpu.CompilerParams(needs_layout_passes=False),
        scratch_types=[
            pltpu.VMEM((FLAT_PER_W,), jnp.float32),
            pltpu.VMEM((BOUNDS_SIZE,), jnp.float32),
            pltpu.VMEM((MS_SIZE,), jnp.float32),
            pltpu.VMEM((MS_SIZE,), jnp.float32),
            pltpu.VMEM((TABLE_SIZE,), jnp.float32),
            pltpu.VMEM((OUT_PER_W,), jnp.float32),
        ],
    )(raw_flat, bounds_b, means_b, stds_b, table_flat)


def kernel(raw_features, feature_means, feature_stds, bin_boundaries,
           emb_tables):
    raw_flat = raw_features.reshape(BATCH * NUM_FEATURES)
    inner = bin_boundaries[:, 1:-1]
    bounds_b = jnp.broadcast_to(
        inner[:, :, None], (NUM_FEATURES, NUM_INNER, LANES)).reshape(-1)
    means_b = jnp.broadcast_to(
        feature_means[:, None], (NUM_FEATURES, LANES)).reshape(-1)
    stds_b = jnp.broadcast_to(
        feature_stds[:, None], (NUM_FEATURES, LANES)).reshape(-1)
    table_flat = emb_tables.reshape(TABLE_SIZE)
    out = _encode(raw_flat, bounds_b, means_b, stds_b, table_flat)
    return out.reshape(BATCH, NUM_FEATURES * EMB_DIM)


# speedup vs baseline: 1.2785x; 1.2785x over previous
import jax
import jax.numpy as jnp
from jax import lax
from jax.experimental import pallas as pl
from jax.experimental.pallas import tpu as pltpu
from jax.experimental.pallas import tpu_sc as plsc

NUM_FEATURES = 7
NUM_BINS = 10
EMB_DIM = 18
BATCH = 16384
NUM_INNER = NUM_BINS - 1

NC = 2
NS = 16
NW = NC * NS
LANES = 16

ROWS_PER_W = BATCH // NW
FLAT_PER_W = ROWS_PER_W * NUM_FEATURES
OUT_PER_W = FLAT_PER_W * EMB_DIM
GROUPS = ROWS_PER_W // LANES

BOUNDS_OFF = 0
BOUNDS_SIZE = NUM_FEATURES * NUM_INNER * LANES
MEANS_OFF = BOUNDS_OFF + BOUNDS_SIZE
MS_SIZE = NUM_FEATURES * LANES
STDS_OFF = MEANS_OFF + MS_SIZE
TABLE_OFF = STDS_OFF + MS_SIZE
TABLE_SIZE = NUM_FEATURES * NUM_BINS * EMB_DIM
CONST_SIZE = ((TABLE_OFF + TABLE_SIZE + 7) // 8) * 8


def _sc_body(raw_hbm, const_hbm, out_hbm, raw_v, const_v, rows_v):
    wid = lax.axis_index("s") * NC + lax.axis_index("c")
    base = wid * FLAT_PER_W

    pltpu.sync_copy(raw_hbm.at[pl.ds(base, FLAT_PER_W)], raw_v)
    pltpu.sync_copy(const_hbm, const_v)

    lane = lax.broadcasted_iota(jnp.int32, (LANES,), 0)
    lane_f = lane * NUM_FEATURES
    lane_fe = lane_f * EMB_DIM

    for i in range(NUM_FEATURES):
        mean_i = const_v[pl.ds(MEANS_OFF + i * LANES, LANES)]
        std_i = const_v[pl.ds(STDS_OFF + i * LANES, LANES)]
        bvecs = [const_v[pl.ds(BOUNDS_OFF + (i * NUM_INNER + k) * LANES,
                               LANES)]
                 for k in range(NUM_INNER)]

        def body(g, carry, i=i, mean_i=mean_i, std_i=std_i, bvecs=bvecs):
            flat0 = g * (LANES * NUM_FEATURES) + i
            gidx = lane_f + flat0
            vals = plsc.load_gather(raw_v, [gidx])
            x = (vals - mean_i) / (std_i + 1e-8)
            cnt = jnp.zeros((LANES,), jnp.int32)
            for k in range(NUM_INNER):
                cnt = cnt + (x > bvecs[k]).astype(jnp.int32)
            src0 = cnt * EMB_DIM + (TABLE_OFF + i * NUM_BINS * EMB_DIM)
            dst0 = lane_fe + flat0 * EMB_DIM
            es = [plsc.load_gather(const_v, [src0 + d])
                  for d in range(EMB_DIM)]
            for d in range(EMB_DIM):
                plsc.store_scatter(rows_v, [dst0 + d], es[d])
            return carry

        lax.fori_loop(0, GROUPS, body, 0)

    pltpu.sync_copy(rows_v, out_hbm.at[pl.ds(wid * OUT_PER_W, OUT_PER_W)])


@jax.jit
def _encode(raw_flat, const_b):
    mesh = plsc.VectorSubcoreMesh(
        core_axis_name="c", subcore_axis_name="s",
        num_cores=NC, num_subcores=NS,
    )
    return pl.kernel(
        _sc_body,
        out_type=jax.ShapeDtypeStruct((BATCH * NUM_FEATURES * EMB_DIM,),
                                      jnp.float32),
        mesh=mesh,
        compiler_params=pltpu.CompilerParams(needs_layout_passes=False),
        scratch_types=[
            pltpu.VMEM((FLAT_PER_W,), jnp.float32),
            pltpu.VMEM((CONST_SIZE,), jnp.float32),
            pltpu.VMEM((OUT_PER_W,), jnp.float32),
        ],
    )(raw_flat, const_b)


def kernel(raw_features, feature_means, feature_stds, bin_boundaries,
           emb_tables):
    raw_flat = raw_features.reshape(BATCH * NUM_FEATURES)
    inner = bin_boundaries[:, 1:-1]
    bounds_b = jnp.broadcast_to(
        inner[:, :, None], (NUM_FEATURES, NUM_INNER, LANES)).reshape(-1)
    means_b = jnp.broadcast_to(
        feature_means[:, None], (NUM_FEATURES, LANES)).reshape(-1)
    stds_b = jnp.broadcast_to(
        feature_stds[:, None], (NUM_FEATURES, LANES)).reshape(-1)
    table_flat = emb_tables.reshape(TABLE_SIZE)
    const_b = jnp.concatenate([
        bounds_b, means_b, stds_b, table_flat,
        jnp.zeros((CONST_SIZE - TABLE_OFF - TABLE_SIZE,), jnp.float32),
    ])
    out = _encode(raw_flat, const_b)
    return out.reshape(BATCH, NUM_FEATURES * EMB_DIM)


# speedup vs baseline: 1.7390x; 1.3601x over previous
import functools

import jax
import jax.numpy as jnp
from jax import lax
from jax.experimental import pallas as pl
from jax.experimental.pallas import tpu as pltpu
from jax.experimental.pallas import tpu_sc as plsc

NUM_FEATURES = 7
NUM_BINS = 10
EMB_DIM = 18
BATCH = 16384
NUM_INNER = NUM_BINS - 1
OUT_PITCH = 128

NC = 2
NS = 16
NW = NC * NS
LANES = 16

ROWS_PER_W = BATCH // NW
OUT_PER_W = ROWS_PER_W * OUT_PITCH
GROUPS = ROWS_PER_W // LANES

BOUNDS_OFF = 0
BOUNDS_SIZE = NUM_FEATURES * NUM_INNER * LANES
MEANS_OFF = BOUNDS_OFF + BOUNDS_SIZE
MS_SIZE = NUM_FEATURES * LANES
STDS_OFF = MEANS_OFF + MS_SIZE
TABLE_OFF = STDS_OFF + MS_SIZE
TABLE_SIZE = NUM_FEATURES * NUM_BINS * EMB_DIM
CONST_SIZE = ((TABLE_OFF + TABLE_SIZE + 7) // 8) * 8

def _take(a, idx):
    return a.at[idx].get(mode="promise_in_bounds")


def _sc_body(raw_hbm, const_hbm, out_hbm, raw_v, const_v, rows_v, sem):
    wid = lax.axis_index("s") * NC + lax.axis_index("c")

    descs = [
        pltpu.async_copy(
            raw_hbm.at[pl.ds(i * BATCH + wid * ROWS_PER_W, ROWS_PER_W)],
            raw_v.at[pl.ds(i * ROWS_PER_W, ROWS_PER_W)], sem)
        for i in range(NUM_FEATURES)
    ]
    descs.append(pltpu.async_copy(const_hbm, const_v, sem))
    for d in descs:
        d.wait()

    lane = lax.broadcasted_iota(jnp.int32, (LANES,), 0)
    full_l = [jnp.full((LANES,), l, jnp.int32) for l in range(LANES)]
    half = lane >> 1
    parity = lane & 1
    tail_e = [half + (j * 8) for j in range(2)]
    tail_src = parity + 16
    tail_dst = [tail_e[j] * OUT_PITCH + tail_src for j in range(2)]

    for i in range(NUM_FEATURES):
        mean_i = const_v[pl.ds(MEANS_OFF + i * LANES, LANES)]
        std_i = const_v[pl.ds(STDS_OFF + i * LANES, LANES)]
        bvecs = [const_v[pl.ds(BOUNDS_OFF + (i * NUM_INNER + k) * LANES,
                               LANES)]
                 for k in range(NUM_INNER)]

        def body(g, carry, i=i, mean_i=mean_i, std_i=std_i, bvecs=bvecs):
            vals = raw_v[pl.ds(i * ROWS_PER_W + g * LANES, LANES)]
            x = (vals - mean_i) / (std_i + 1e-8)
            cs = [(x > bvecs[k]).astype(jnp.int32) for k in range(NUM_INNER)]
            cnt = (((cs[0] + cs[1]) + (cs[2] + cs[3]))
                   + ((cs[4] + cs[5]) + (cs[6] + cs[7]))) + cs[8]
            src0 = cnt * EMB_DIM + (TABLE_OFF + i * NUM_BINS * EMB_DIM)
            gbase = g * (LANES * OUT_PITCH)
            for l in range(LANES):
                si = _take(src0, full_l[l]) + lane
                di = lane + (gbase + (l * OUT_PITCH + i * EMB_DIM))
                e = plsc.load_gather(const_v, [si])
                plsc.store_scatter(rows_v, [di], e)
            for j in range(2):
                si = _take(src0, tail_e[j]) + tail_src
                di = tail_dst[j] + (gbase + i * EMB_DIM)
                e = plsc.load_gather(const_v, [si])
                plsc.store_scatter(rows_v, [di], e)
            return carry

        lax.fori_loop(0, GROUPS, body, 0)

    pltpu.sync_copy(rows_v, out_hbm.at[pl.ds(wid * OUT_PER_W, OUT_PER_W)])


@jax.jit
def _encode(raw_t_flat, const_b):
    mesh = plsc.VectorSubcoreMesh(
        core_axis_name="c", subcore_axis_name="s",
        num_cores=NC, num_subcores=NS,
    )
    return pl.kernel(
        _sc_body,
        out_type=jax.ShapeDtypeStruct((BATCH * OUT_PITCH,), jnp.float32),
        mesh=mesh,
        compiler_params=pltpu.CompilerParams(needs_layout_passes=False),
        scratch_types=[
            pltpu.VMEM((NUM_FEATURES * ROWS_PER_W,), jnp.float32),
            pltpu.VMEM((CONST_SIZE,), jnp.float32),
            pltpu.VMEM((OUT_PER_W,), jnp.float32),
            pltpu.SemaphoreType.DMA,
        ],
    )(raw_t_flat, const_b)


def kernel(raw_features, feature_means, feature_stds, bin_boundaries,
           emb_tables):
    raw_t_flat = raw_features.T.reshape(BATCH * NUM_FEATURES)
    inner = bin_boundaries[:, 1:-1]
    bounds_b = jnp.broadcast_to(
        inner[:, :, None], (NUM_FEATURES, NUM_INNER, LANES)).reshape(-1)
    means_b = jnp.broadcast_to(
        feature_means[:, None], (NUM_FEATURES, LANES)).reshape(-1)
    stds_b = jnp.broadcast_to(
        feature_stds[:, None], (NUM_FEATURES, LANES)).reshape(-1)
    table_flat = emb_tables.reshape(TABLE_SIZE)
    const_b = jnp.concatenate([
        bounds_b, means_b, stds_b, table_flat,
        jnp.zeros((CONST_SIZE - TABLE_OFF - TABLE_SIZE,), jnp.float32),
    ])
    out = _encode(raw_t_flat, const_b)
    return out.reshape(BATCH, OUT_PITCH)[:, :NUM_FEATURES * EMB_DIM]


# speedup vs baseline: 2.4446x; 1.4058x over previous
import jax
import jax.numpy as jnp
from jax import lax
from jax.experimental import pallas as pl
from jax.experimental.pallas import tpu as pltpu
from jax.experimental.pallas import tpu_sc as plsc

NUM_FEATURES = 7
NUM_BINS = 10
EMB_DIM = 18
BATCH = 16384
NUM_INNER = NUM_BINS - 1
OUT_PITCH = 128

NC = 2
NS = 16
NW = NC * NS
LANES = 16

ROWS_PER_W = BATCH // NW
OUT_PER_W = ROWS_PER_W * OUT_PITCH
GROUPS = ROWS_PER_W // LANES
GBLOCK = LANES * OUT_PITCH

INNER_OFF = 0
MEANS_OFF = NUM_FEATURES * NUM_INNER
STDS_OFF = MEANS_OFF + NUM_FEATURES
TABLE_OFF = 80
TABLE_SIZE = NUM_FEATURES * NUM_BINS * EMB_DIM
CONST_SIZE = TABLE_OFF + TABLE_SIZE + 4

EB_OFF = 0
EM_OFF = NUM_FEATURES * NUM_INNER * LANES
ES_OFF = EM_OFF + NUM_FEATURES * LANES
EXP_SIZE = ES_OFF + NUM_FEATURES * LANES


def _take(a, idx):
    return a.at[idx].get(mode="promise_in_bounds")


def _sc_body(raw_hbm, const_hbm, out_hbm, raw_v, const_v, exp_v, rows_v,
             sem, osem):
    wid = lax.axis_index("s") * NC + lax.axis_index("c")
    wbase = wid * OUT_PER_W

    descs = [
        pltpu.async_copy(
            raw_hbm.at[pl.ds(i * BATCH + wid * ROWS_PER_W, ROWS_PER_W)],
            raw_v.at[pl.ds(i * ROWS_PER_W, ROWS_PER_W)], sem)
        for i in range(NUM_FEATURES)
    ]
    descs.append(pltpu.async_copy(const_hbm, const_v, sem))
    for d in descs:
        d.wait()

    lane = lax.broadcasted_iota(jnp.int32, (LANES,), 0)
    full_l = [jnp.full((LANES,), l, jnp.int32) for l in range(LANES)]
    half = lane >> 1
    parity = lane & 1
    tail_e = [half + (j * 8) for j in range(2)]
    tail_src = parity + 16
    tail_dst = [tail_e[j] * OUT_PITCH + tail_src for j in range(2)]

    packs = [const_v[pl.ds(v * LANES, LANES)] for v in range(5)]

    def bcast(p):
        return _take(packs[p // LANES], full_l[p % LANES])

    for i in range(NUM_FEATURES):
        for k in range(NUM_INNER):
            exp_v[pl.ds(EB_OFF + (i * NUM_INNER + k) * LANES, LANES)] = (
                bcast(INNER_OFF + i * NUM_INNER + k))
        exp_v[pl.ds(EM_OFF + i * LANES, LANES)] = bcast(MEANS_OFF + i)
        exp_v[pl.ds(ES_OFF + i * LANES, LANES)] = bcast(STDS_OFF + i)

    def body(g, carry):
        gbase = g * GBLOCK
        for i in range(NUM_FEATURES):
            mean_i = exp_v[pl.ds(EM_OFF + i * LANES, LANES)]
            std_i = exp_v[pl.ds(ES_OFF + i * LANES, LANES)]
            vals = raw_v[pl.ds(i * ROWS_PER_W + g * LANES, LANES)]
            x = (vals - mean_i) / (std_i + 1e-8)
            cs = []
            for k in range(NUM_INNER):
                bk = exp_v[pl.ds(EB_OFF + (i * NUM_INNER + k) * LANES,
                                 LANES)]
                cs.append((x > bk).astype(jnp.int32))
            cnt = (((cs[0] + cs[1]) + (cs[2] + cs[3]))
                   + ((cs[4] + cs[5]) + (cs[6] + cs[7]))) + cs[8]
            src0 = cnt * EMB_DIM + (TABLE_OFF + i * NUM_BINS * EMB_DIM)
            es = [plsc.load_gather(const_v, [_take(src0, full_l[l]) + lane])
                  for l in range(LANES)]
            ts = [plsc.load_gather(const_v, [_take(src0, tail_e[j])
                                             + tail_src])
                  for j in range(2)]
            for l in range(LANES):
                di = lane + (gbase + (l * OUT_PITCH + i * EMB_DIM))
                plsc.store_scatter(rows_v, [di], es[l])
            for j in range(2):
                plsc.store_scatter(rows_v, [tail_dst[j]
                                            + (gbase + i * EMB_DIM)], ts[j])
        pltpu.async_copy(rows_v.at[pl.ds(gbase, GBLOCK)],
                         out_hbm.at[pl.ds(wbase + gbase, GBLOCK)], osem)

        @pl.when(g > 0)
        def _():
            pltpu.make_async_copy(
                rows_v.at[pl.ds(gbase - GBLOCK, GBLOCK)],
                out_hbm.at[pl.ds(wbase + gbase - GBLOCK, GBLOCK)],
                osem).wait()

        return carry

    lax.fori_loop(0, GROUPS, body, 0)
    pltpu.make_async_copy(
        rows_v.at[pl.ds((GROUPS - 1) * GBLOCK, GBLOCK)],
        out_hbm.at[pl.ds(wbase + (GROUPS - 1) * GBLOCK, GBLOCK)],
        osem).wait()


@jax.jit
def _encode(raw_t_flat, const_b):
    mesh = plsc.VectorSubcoreMesh(
        core_axis_name="c", subcore_axis_name="s",
        num_cores=NC, num_subcores=NS,
    )
    return pl.kernel(
        _sc_body,
        out_type=jax.ShapeDtypeStruct((BATCH * OUT_PITCH,), jnp.float32),
        mesh=mesh,
        compiler_params=pltpu.CompilerParams(needs_layout_passes=False),
        scratch_types=[
            pltpu.VMEM((NUM_FEATURES * ROWS_PER_W,), jnp.float32),
            pltpu.VMEM((CONST_SIZE,), jnp.float32),
            pltpu.VMEM((EXP_SIZE,), jnp.float32),
            pltpu.VMEM((OUT_PER_W,), jnp.float32),
            pltpu.SemaphoreType.DMA,
            pltpu.SemaphoreType.DMA,
        ],
    )(raw_t_flat, const_b)


def kernel(raw_features, feature_means, feature_stds, bin_boundaries,
           emb_tables):
    raw_t_flat = raw_features.T.reshape(BATCH * NUM_FEATURES)
    inner_flat = bin_boundaries[:, 1:-1].reshape(-1)
    table_flat = emb_tables.reshape(TABLE_SIZE)
    const_b = jnp.concatenate([
        inner_flat, feature_means, feature_stds,
        jnp.zeros((TABLE_OFF - STDS_OFF - NUM_FEATURES,), jnp.float32),
        table_flat,
        jnp.zeros((CONST_SIZE - TABLE_OFF - TABLE_SIZE,), jnp.float32),
    ])
    out = _encode(raw_t_flat, const_b)
    return out.reshape(BATCH, OUT_PITCH)[:, :NUM_FEATURES * EMB_DIM]


# speedup vs baseline: 2.4465x; 1.0008x over previous
import jax
import jax.numpy as jnp
from jax import lax
from jax.experimental import pallas as pl
from jax.experimental.pallas import tpu as pltpu
from jax.experimental.pallas import tpu_sc as plsc

NUM_FEATURES = 7
NUM_BINS = 10
EMB_DIM = 18
BATCH = 16384
NUM_INNER = NUM_BINS - 1
OUT_PITCH = 128

NC = 2
NS = 16
NW = NC * NS
LANES = 16

ROWS_PER_W = BATCH // NW
OUT_PER_W = ROWS_PER_W * OUT_PITCH
GROUPS = ROWS_PER_W // LANES
GBLOCK = LANES * OUT_PITCH

INNER_OFF = 0
MEANS_OFF = NUM_FEATURES * NUM_INNER
STDS_OFF = MEANS_OFF + NUM_FEATURES
TABLE_OFF = 80
TABLE_SIZE = NUM_FEATURES * NUM_BINS * EMB_DIM
CONST_SIZE = TABLE_OFF + TABLE_SIZE + 4

EB_OFF = 0
EM_OFF = NUM_FEATURES * NUM_INNER * LANES
ES_OFF = EM_OFF + NUM_FEATURES * LANES
EXP_SIZE = ES_OFF + NUM_FEATURES * LANES


def _take(a, idx):
    return a.at[idx].get(mode="promise_in_bounds")


def _sc_body(raw_hbm, const_hbm, out_hbm, raw_v, const_v, exp_v, rows_v,
             sem, osem):
    wid = lax.axis_index("s") * NC + lax.axis_index("c")
    wbase = wid * OUT_PER_W

    descs = [
        pltpu.async_copy(
            raw_hbm.at[pl.ds(i * BATCH + wid * ROWS_PER_W, ROWS_PER_W)],
            raw_v.at[pl.ds(i * ROWS_PER_W, ROWS_PER_W)], sem)
        for i in range(NUM_FEATURES)
    ]
    descs.append(pltpu.async_copy(const_hbm, const_v, sem))
    for d in descs:
        d.wait()

    lane = lax.broadcasted_iota(jnp.int32, (LANES,), 0)
    full_l = [jnp.full((LANES,), l, jnp.int32) for l in range(LANES)]
    half = lane >> 1
    parity = lane & 1
    tail_e = [half + (j * 8) for j in range(2)]
    tail_src = parity + 16
    tail_dst = [tail_e[j] * OUT_PITCH + tail_src for j in range(2)]

    packs = [const_v[pl.ds(v * LANES, LANES)] for v in range(5)]

    def bcast(p):
        return _take(packs[p // LANES], full_l[p % LANES])

    for i in range(NUM_FEATURES):
        for k in range(NUM_INNER):
            exp_v[pl.ds(EB_OFF + (i * NUM_INNER + k) * LANES, LANES)] = (
                bcast(INNER_OFF + i * NUM_INNER + k))
        exp_v[pl.ds(EM_OFF + i * LANES, LANES)] = bcast(MEANS_OFF + i)
        exp_v[pl.ds(ES_OFF + i * LANES, LANES)] = bcast(STDS_OFF + i)

    def body(g, carry):
        gbase = g * GBLOCK
        for i in range(NUM_FEATURES):
            mean_i = exp_v[pl.ds(EM_OFF + i * LANES, LANES)]
            std_i = exp_v[pl.ds(ES_OFF + i * LANES, LANES)]
            vals = raw_v[pl.ds(i * ROWS_PER_W + g * LANES, LANES)]
            x = (vals - mean_i) / (std_i + 1e-8)
            cs = []
            for k in range(NUM_INNER):
                bk = exp_v[pl.ds(EB_OFF + (i * NUM_INNER + k) * LANES,
                                 LANES)]
                cs.append((x > bk).astype(jnp.int32))
            cnt = (((cs[0] + cs[1]) + (cs[2] + cs[3]))
                   + ((cs[4] + cs[5]) + (cs[6] + cs[7]))) + cs[8]
            src0 = cnt * EMB_DIM + (TABLE_OFF + i * NUM_BINS * EMB_DIM)
            es = [plsc.load_gather(const_v, [_take(src0, full_l[l]) + lane])
                  for l in range(LANES)]
            ts = [plsc.load_gather(const_v, [_take(src0, tail_e[j])
                                             + tail_src])
                  for j in range(2)]
            for l in range(LANES):
                rows_v[pl.ds(gbase + (l * OUT_PITCH + i * EMB_DIM),
                             LANES)] = es[l]
            for j in range(2):
                plsc.store_scatter(rows_v, [tail_dst[j]
                                            + (gbase + i * EMB_DIM)], ts[j])
        pltpu.async_copy(rows_v.at[pl.ds(gbase, GBLOCK)],
                         out_hbm.at[pl.ds(wbase + gbase, GBLOCK)], osem)

        @pl.when(g > 0)
        def _():
            pltpu.make_async_copy(
                rows_v.at[pl.ds(gbase - GBLOCK, GBLOCK)],
                out_hbm.at[pl.ds(wbase + gbase - GBLOCK, GBLOCK)],
                osem).wait()

        return carry

    lax.fori_loop(0, GROUPS, body, 0)
    pltpu.make_async_copy(
        rows_v.at[pl.ds((GROUPS - 1) * GBLOCK, GBLOCK)],
        out_hbm.at[pl.ds(wbase + (GROUPS - 1) * GBLOCK, GBLOCK)],
        osem).wait()


@jax.jit
def _encode(raw_t_flat, const_b):
    mesh = plsc.VectorSubcoreMesh(
        core_axis_name="c", subcore_axis_name="s",
        num_cores=NC, num_subcores=NS,
    )
    return pl.kernel(
        _sc_body,
        out_type=jax.ShapeDtypeStruct((BATCH * OUT_PITCH,), jnp.float32),
        mesh=mesh,
        compiler_params=pltpu.CompilerParams(needs_layout_passes=False),
        scratch_types=[
            pltpu.VMEM((NUM_FEATURES * ROWS_PER_W,), jnp.float32),
            pltpu.VMEM((CONST_SIZE,), jnp.float32),
            pltpu.VMEM((EXP_SIZE,), jnp.float32),
            pltpu.VMEM((OUT_PER_W,), jnp.float32),
            pltpu.SemaphoreType.DMA,
            pltpu.SemaphoreType.DMA,
        ],
    )(raw_t_flat, const_b)


def kernel(raw_features, feature_means, feature_stds, bin_boundaries,
           emb_tables):
    raw_t_flat = raw_features.T.reshape(BATCH * NUM_FEATURES)
    inner_flat = bin_boundaries[:, 1:-1].reshape(-1)
    table_flat = emb_tables.reshape(TABLE_SIZE)
    const_b = jnp.concatenate([
        inner_flat, feature_means, feature_stds,
        jnp.zeros((TABLE_OFF - STDS_OFF - NUM_FEATURES,), jnp.float32),
        table_flat,
        jnp.zeros((CONST_SIZE - TABLE_OFF - TABLE_SIZE,), jnp.float32),
    ])
    out = _encode(raw_t_flat, const_b)
    return out.reshape(BATCH, OUT_PITCH)[:, :NUM_FEATURES * EMB_DIM]


# speedup vs baseline: 2.4481x; 1.0007x over previous
import jax
import jax.numpy as jnp
from jax import lax
from jax.experimental import pallas as pl
from jax.experimental.pallas import tpu as pltpu
from jax.experimental.pallas import tpu_sc as plsc

NUM_FEATURES = 7
NUM_BINS = 10
EMB_DIM = 18
BATCH = 16384
NUM_INNER = NUM_BINS - 1
OUT_PITCH = 128

NC = 2
NS = 16
NW = NC * NS
LANES = 16

ROWS_PER_W = BATCH // NW
OUT_PER_W = ROWS_PER_W * OUT_PITCH
GROUPS = ROWS_PER_W // LANES
GBLOCK = LANES * OUT_PITCH

INNER_OFF = 0
MEANS_OFF = NUM_FEATURES * NUM_INNER
STDS_OFF = MEANS_OFF + NUM_FEATURES
TABLE_OFF = 80
TABLE_SIZE = NUM_FEATURES * NUM_BINS * EMB_DIM
CONST_SIZE = TABLE_OFF + TABLE_SIZE + 4

EB_OFF = 0
EM_OFF = NUM_FEATURES * NUM_INNER * LANES
ES_OFF = EM_OFF + NUM_FEATURES * LANES
EXP_SIZE = ES_OFF + NUM_FEATURES * LANES


def _take(a, idx):
    return a.at[idx].get(mode="promise_in_bounds")


def _sc_body(raw_hbm, const_hbm, out_hbm, raw_v, const_v, exp_v, rows_v,
             sem, osem):
    wid = lax.axis_index("s") * NC + lax.axis_index("c")
    wbase = wid * OUT_PER_W

    descs = [
        pltpu.async_copy(
            raw_hbm.at[pl.ds(i * BATCH + wid * ROWS_PER_W, ROWS_PER_W)],
            raw_v.at[pl.ds(i * ROWS_PER_W, ROWS_PER_W)], sem)
        for i in range(NUM_FEATURES)
    ]
    descs.append(pltpu.async_copy(const_hbm, const_v, sem))
    for d in descs:
        d.wait()

    lane = lax.broadcasted_iota(jnp.int32, (LANES,), 0)
    full_l = [jnp.full((LANES,), l, jnp.int32) for l in range(LANES)]
    half = lane >> 1
    parity = lane & 1
    tail_e = [half + (j * 8) for j in range(2)]
    tail_src = parity + 16
    tail_dst = [tail_e[j] * OUT_PITCH + tail_src for j in range(2)]

    packs = [const_v[pl.ds(v * LANES, LANES)] for v in range(5)]

    def bcast(p):
        return _take(packs[p // LANES], full_l[p % LANES])

    for i in range(NUM_FEATURES):
        for k in range(NUM_INNER):
            exp_v[pl.ds(EB_OFF + (i * NUM_INNER + k) * LANES, LANES)] = (
                bcast(INNER_OFF + i * NUM_INNER + k))
        exp_v[pl.ds(EM_OFF + i * LANES, LANES)] = bcast(MEANS_OFF + i)
        exp_v[pl.ds(ES_OFF + i * LANES, LANES)] = bcast(STDS_OFF + i)

    def body(g, carry):
        gbase = g * GBLOCK

        def feat(i, c):
            mean_i = exp_v[pl.ds(EM_OFF + i * LANES, LANES)]
            std_i = exp_v[pl.ds(ES_OFF + i * LANES, LANES)]
            vals = raw_v[pl.ds(i * ROWS_PER_W + g * LANES, LANES)]
            x = (vals - mean_i) / (std_i + 1e-8)
            cs = []
            for k in range(NUM_INNER):
                bk = exp_v[pl.ds(EB_OFF + i * (NUM_INNER * LANES)
                                 + k * LANES, LANES)]
                cs.append((x > bk).astype(jnp.int32))
            cnt = (((cs[0] + cs[1]) + (cs[2] + cs[3]))
                   + ((cs[4] + cs[5]) + (cs[6] + cs[7]))) + cs[8]
            src0 = cnt * EMB_DIM + (i * (NUM_BINS * EMB_DIM) + TABLE_OFF)
            es = [plsc.load_gather(const_v, [_take(src0, full_l[l]) + lane])
                  for l in range(LANES)]
            ts = [plsc.load_gather(const_v, [_take(src0, tail_e[j])
                                             + tail_src])
                  for j in range(2)]
            dbase = gbase + i * EMB_DIM
            for l in range(LANES):
                plsc.store_scatter(rows_v, [lane + (dbase + l * OUT_PITCH)],
                                   es[l])
            for j in range(2):
                plsc.store_scatter(rows_v, [tail_dst[j] + dbase], ts[j])
            return c

        lax.fori_loop(0, NUM_FEATURES, feat, 0)
        pltpu.async_copy(rows_v.at[pl.ds(gbase, GBLOCK)],
                         out_hbm.at[pl.ds(wbase + gbase, GBLOCK)], osem)

        @pl.when(g > 0)
        def _():
            pltpu.make_async_copy(
                rows_v.at[pl.ds(gbase - GBLOCK, GBLOCK)],
                out_hbm.at[pl.ds(wbase + gbase - GBLOCK, GBLOCK)],
                osem).wait()

        return carry

    lax.fori_loop(0, GROUPS, body, 0)
    pltpu.make_async_copy(
        rows_v.at[pl.ds((GROUPS - 1) * GBLOCK, GBLOCK)],
        out_hbm.at[pl.ds(wbase + (GROUPS - 1) * GBLOCK, GBLOCK)],
        osem).wait()


@jax.jit
def _encode(raw_t_flat, const_b):
    mesh = plsc.VectorSubcoreMesh(
        core_axis_name="c", subcore_axis_name="s",
        num_cores=NC, num_subcores=NS,
    )
    return pl.kernel(
        _sc_body,
        out_type=jax.ShapeDtypeStruct((BATCH * OUT_PITCH,), jnp.float32),
        mesh=mesh,
        compiler_params=pltpu.CompilerParams(needs_layout_passes=False),
        scratch_types=[
            pltpu.VMEM((NUM_FEATURES * ROWS_PER_W,), jnp.float32),
            pltpu.VMEM((CONST_SIZE,), jnp.float32),
            pltpu.VMEM((EXP_SIZE,), jnp.float32),
            pltpu.VMEM((OUT_PER_W,), jnp.float32),
            pltpu.SemaphoreType.DMA,
            pltpu.SemaphoreType.DMA,
        ],
    )(raw_t_flat, const_b)


def kernel(raw_features, feature_means, feature_stds, bin_boundaries,
           emb_tables):
    raw_t_flat = raw_features.T.reshape(BATCH * NUM_FEATURES)
    inner_flat = bin_boundaries[:, 1:-1].reshape(-1)
    table_flat = emb_tables.reshape(TABLE_SIZE)
    const_b = jnp.concatenate([
        inner_flat, feature_means, feature_stds,
        jnp.zeros((TABLE_OFF - STDS_OFF - NUM_FEATURES,), jnp.float32),
        table_flat,
        jnp.zeros((CONST_SIZE - TABLE_OFF - TABLE_SIZE,), jnp.float32),
    ])
    out = _encode(raw_t_flat, const_b)
    return out.reshape(BATCH, OUT_PITCH)[:, :NUM_FEATURES * EMB_DIM]


# speedup vs baseline: 2.4692x; 1.0086x over previous
import jax
import jax.numpy as jnp
from jax import lax
from jax.experimental import pallas as pl
from jax.experimental.pallas import tpu as pltpu
from jax.experimental.pallas import tpu_sc as plsc

NUM_FEATURES = 7
NUM_BINS = 10
EMB_DIM = 18
BATCH = 16384
NUM_INNER = NUM_BINS - 1
OUT_PITCH = 128

NC = 2
NS = 16
NW = NC * NS
LANES = 16

ROWS_PER_W = BATCH // NW
OUT_PER_W = ROWS_PER_W * OUT_PITCH
GROUPS = ROWS_PER_W // LANES
GBLOCK = LANES * OUT_PITCH

TABLE_SIZE = NUM_FEATURES * NUM_BINS * EMB_DIM
BB_SIZE = NUM_FEATURES * (NUM_BINS + 1)

EB_OFF = 0
EM_OFF = NUM_FEATURES * NUM_INNER * LANES
ES_OFF = EM_OFF + NUM_FEATURES * LANES
EXP_SIZE = ES_OFF + NUM_FEATURES * LANES


def _take(a, idx):
    return a.at[idx].get(mode="promise_in_bounds")


def _sc_body(raw_hbm, bb_hbm, means_hbm, stds_hbm, table_hbm, out_hbm,
             raw_v, bb_v, ms_v, table_v, exp_v, rows_v, sem, osem):
    wid = lax.axis_index("s") * NC + lax.axis_index("c")
    wbase = wid * OUT_PER_W

    descs = [
        pltpu.async_copy(
            raw_hbm.at[pl.ds(i * BATCH + wid * ROWS_PER_W, ROWS_PER_W)],
            raw_v.at[pl.ds(i * ROWS_PER_W, ROWS_PER_W)], sem)
        for i in range(NUM_FEATURES)
    ]
    descs.append(pltpu.async_copy(bb_hbm, bb_v.at[pl.ds(0, BB_SIZE)], sem))
    descs.append(pltpu.async_copy(means_hbm, ms_v.at[pl.ds(0, NUM_FEATURES)],
                                  sem))
    descs.append(pltpu.async_copy(stds_hbm,
                                  ms_v.at[pl.ds(LANES, NUM_FEATURES)], sem))
    descs.append(pltpu.async_copy(table_hbm, table_v.at[pl.ds(0, TABLE_SIZE)],
                                  sem))
    for d in descs:
        d.wait()

    lane = lax.broadcasted_iota(jnp.int32, (LANES,), 0)
    full_l = [jnp.full((LANES,), l, jnp.int32) for l in range(LANES)]
    half = lane >> 1
    parity = lane & 1
    tail_e = [half + (j * 8) for j in range(2)]
    tail_src = parity + 16
    tail_dst = [tail_e[j] * OUT_PITCH + tail_src for j in range(2)]

    packs = [bb_v[pl.ds(v * LANES, LANES)] for v in range(5)]
    pm = ms_v[pl.ds(0, LANES)]
    ps = ms_v[pl.ds(LANES, LANES)]

    def bcast(p):
        return _take(packs[p // LANES], full_l[p % LANES])

    for i in range(NUM_FEATURES):
        for k in range(NUM_INNER):
            exp_v[pl.ds(EB_OFF + (i * NUM_INNER + k) * LANES, LANES)] = (
                bcast(i * (NUM_BINS + 1) + k + 1))
        exp_v[pl.ds(EM_OFF + i * LANES, LANES)] = _take(pm, full_l[i])
        exp_v[pl.ds(ES_OFF + i * LANES, LANES)] = _take(ps, full_l[i])

    def body(g, carry):
        gbase = g * GBLOCK

        def feat(i, c):
            mean_i = exp_v[pl.ds(EM_OFF + i * LANES, LANES)]
            std_i = exp_v[pl.ds(ES_OFF + i * LANES, LANES)]
            vals = raw_v[pl.ds(i * ROWS_PER_W + g * LANES, LANES)]
            x = (vals - mean_i) / (std_i + 1e-8)
            cs = []
            for k in range(NUM_INNER):
                bk = exp_v[pl.ds(EB_OFF + i * (NUM_INNER * LANES)
                                 + k * LANES, LANES)]
                cs.append((x > bk).astype(jnp.int32))
            cnt = (((cs[0] + cs[1]) + (cs[2] + cs[3]))
                   + ((cs[4] + cs[5]) + (cs[6] + cs[7]))) + cs[8]
            src0 = cnt * EMB_DIM + i * (NUM_BINS * EMB_DIM)
            es = [plsc.load_gather(table_v, [_take(src0, full_l[l]) + lane])
                  for l in range(LANES)]
            ts = [plsc.load_gather(table_v, [_take(src0, tail_e[j])
                                             + tail_src])
                  for j in range(2)]
            dbase = gbase + i * EMB_DIM
            for l in range(LANES):
                plsc.store_scatter(rows_v, [lane + (dbase + l * OUT_PITCH)],
                                   es[l])
            for j in range(2):
                plsc.store_scatter(rows_v, [tail_dst[j] + dbase], ts[j])
            return c

        lax.fori_loop(0, NUM_FEATURES, feat, 0)
        pltpu.async_copy(rows_v.at[pl.ds(gbase, GBLOCK)],
                         out_hbm.at[pl.ds(wbase + gbase, GBLOCK)], osem)

        @pl.when(g > 0)
        def _():
            pltpu.make_async_copy(
                rows_v.at[pl.ds(gbase - GBLOCK, GBLOCK)],
                out_hbm.at[pl.ds(wbase + gbase - GBLOCK, GBLOCK)],
                osem).wait()

        return carry

    lax.fori_loop(0, GROUPS, body, 0)
    pltpu.make_async_copy(
        rows_v.at[pl.ds((GROUPS - 1) * GBLOCK, GBLOCK)],
        out_hbm.at[pl.ds(wbase + (GROUPS - 1) * GBLOCK, GBLOCK)],
        osem).wait()


@jax.jit
def _encode(raw_t_flat, bb_flat, means, stds, table_flat):
    mesh = plsc.VectorSubcoreMesh(
        core_axis_name="c", subcore_axis_name="s",
        num_cores=NC, num_subcores=NS,
    )
    return pl.kernel(
        _sc_body,
        out_type=jax.ShapeDtypeStruct((BATCH * OUT_PITCH,), jnp.float32),
        mesh=mesh,
        compiler_params=pltpu.CompilerParams(needs_layout_passes=False),
        scratch_types=[
            pltpu.VMEM((NUM_FEATURES * ROWS_PER_W,), jnp.float32),
            pltpu.VMEM((5 * LANES,), jnp.float32),
            pltpu.VMEM((2 * LANES,), jnp.float32),
            pltpu.VMEM((TABLE_SIZE + 4,), jnp.float32),
            pltpu.VMEM((EXP_SIZE,), jnp.float32),
            pltpu.VMEM((OUT_PER_W,), jnp.float32),
            pltpu.SemaphoreType.DMA,
            pltpu.SemaphoreType.DMA,
        ],
    )(raw_t_flat, bb_flat, means, stds, table_flat)


def kernel(raw_features, feature_means, feature_stds, bin_boundaries,
           emb_tables):
    raw_t_flat = raw_features.T.reshape(BATCH * NUM_FEATURES)
    bb_flat = bin_boundaries.reshape(BB_SIZE)
    table_flat = emb_tables.reshape(TABLE_SIZE)
    out = _encode(raw_t_flat, bb_flat, feature_means, feature_stds,
                  table_flat)
    return out.reshape(BATCH, OUT_PITCH)[:, :NUM_FEATURES * EMB_DIM]
